# baseline (speedup 1.0000x reference)
import functools

import jax
import jax.numpy as jnp
from jax import lax
from jax.experimental import pallas as pl
from jax.experimental.pallas import tpu as pltpu
from jax.experimental.pallas import tpu_sc as plsc

BATCH = 16384
DIM = 64
NW = 32
BPW = BATCH // NW
CHUNK = 128
NCH = BPW // CHUNK
G = 16


def _nrsqrt(x):
    i = plsc.bitcast(x, jnp.int32)
    i = jnp.int32(0x5F3759DF) - lax.shift_right_arithmetic(i, jnp.int32(1))
    y = plsc.bitcast(i, jnp.float32)
    for _ in range(3):
        y = y * (1.5 - 0.5 * x * y * y)
    return y


def _body(heads_r, rels_r, tails_r, entity_hbm, relation_hbm, out_hbm,
          idx_h, idx_r, idx_t, h_rows, r_rows, t_rows,
          h_t, r_t, t_t, out_v, sem):
    wid = lax.axis_index("s") * 2 + lax.axis_index("c")
    base = pl.multiple_of(wid * BPW, BPW)
    irow = pl.multiple_of(wid * NCH, NCH)

    pltpu.sync_copy(heads_r.at[pl.ds(irow, NCH)], idx_h)
    pltpu.sync_copy(rels_r.at[pl.ds(irow, NCH)], idx_r)
    pltpu.sync_copy(tails_r.at[pl.ds(irow, NCH)], idx_t)

    copies = []
    for j in range(NCH):
        copies.append(pltpu.async_copy(
            entity_hbm.at[idx_h.at[j]], h_rows.at[pl.ds(j * CHUNK, CHUNK)], sem))
        copies.append(pltpu.async_copy(
            entity_hbm.at[idx_t.at[j]], t_rows.at[pl.ds(j * CHUNK, CHUNK)], sem))
        copies.append(pltpu.async_copy(
            relation_hbm.at[idx_r.at[j]], r_rows.at[pl.ds(j * CHUNK, CHUNK)], sem))
    for cp in copies:
        cp.wait()

    lane = lax.iota(jnp.int32, 16)
    lane16 = lane * 16
    zero = jnp.zeros((16,), jnp.float32)

    def group(g, carry):
        rbase = pl.multiple_of(g * G, G)
        for src, dst in ((h_rows, h_t), (t_rows, t_t), (r_rows, r_t)):
            for i in range(G):
                row = rbase + i
                for k in range(DIM // 16):
                    v = src[row, pl.ds(k * 16, 16)]
                    plsc.store_scatter(dst, [lane16 + (k * 256 + i)], v)
        hh = tt = rr = hr = ht = rt = zero
        for j in range(DIM):
            h = h_t[pl.ds(j * 16, 16)]
            t = t_t[pl.ds(j * 16, 16)]
            r = r_t[pl.ds(j * 16, 16)]
            hh = hh + h * h
            tt = tt + t * t
            rr = rr + r * r
            hr = hr + h * r
            ht = ht + h * t
            rt = rt + r * t
        a = _nrsqrt(jnp.maximum(hh, 1e-24))
        b = _nrsqrt(jnp.maximum(tt, 1e-24))
        s2 = hh * a * a + rr + tt * b * b + 2.0 * (hr * a - ht * (a * b) - rt * b)
        s2 = jnp.maximum(s2, 0.0)
        score = s2 * _nrsqrt(jnp.maximum(s2, 1e-30))
        out_v[pl.ds(rbase, 16)] = score
        return carry

    lax.fori_loop(0, BPW // G, group, 0)
    pltpu.sync_copy(out_v, out_hbm.at[pl.ds(base, BPW)])


_sc_kernel = functools.partial(
    pl.kernel,
    mesh=plsc.VectorSubcoreMesh(core_axis_name="c", subcore_axis_name="s"),
    compiler_params=pltpu.CompilerParams(
        needs_layout_passes=False, use_tc_tiling_on_sc=False),
    out_type=jax.ShapeDtypeStruct((BATCH,), jnp.float32),
    scratch_types=[
        pltpu.VMEM((NCH, CHUNK), jnp.int32),
        pltpu.VMEM((NCH, CHUNK), jnp.int32),
        pltpu.VMEM((NCH, CHUNK), jnp.int32),
        pltpu.VMEM((BPW, DIM), jnp.float32),
        pltpu.VMEM((BPW, DIM), jnp.float32),
        pltpu.VMEM((BPW, DIM), jnp.float32),
        pltpu.VMEM((G * DIM,), jnp.float32),
        pltpu.VMEM((G * DIM,), jnp.float32),
        pltpu.VMEM((G * DIM,), jnp.float32),
        pltpu.VMEM((BPW,), jnp.float32),
        pltpu.SemaphoreType.DMA,
    ],
)(_body)


def kernel(heads, relations, tails, entity_table, relation_table):
    heads_r = heads.astype(jnp.int32).reshape(BATCH // CHUNK, CHUNK)
    rels_r = relations.astype(jnp.int32).reshape(BATCH // CHUNK, CHUNK)
    tails_r = tails.astype(jnp.int32).reshape(BATCH // CHUNK, CHUNK)
    return _sc_kernel(heads_r, rels_r, tails_r, entity_table, relation_table)
